# baseline (speedup 1.0000x reference)
import jax
import jax.numpy as jnp
from jax import lax
from jax.experimental import pallas as pl
from jax.experimental.pallas import tpu as pltpu
from jax.experimental.pallas import tpu_sc as plsc

_NF = 26
_CARD = 100000
_EMB = 32
_BATCH = 16384
_HB = 8192
_GRP = _HB // 16


def _sc_body(xT_hbm, tT_hbm, biasx_hbm, out_hbm, seg_v, idx_v, out_v, bias_v):
    c = lax.axis_index("s") * 2 + lax.axis_index("c")
    pltpu.sync_copy(biasx_hbm.at[c], bias_v)

    for f in range(_NF):
        pltpu.sync_copy(tT_hbm.at[c, pl.ds(f * _CARD, _CARD)], seg_v)
        bvec = bias_v[pl.ds(f * 16, 16)]

        for h in range(2):
            pltpu.sync_copy(xT_hbm.at[f, pl.ds(h * _HB, _HB)], idx_v)

            def grp(g, carry):
                s = pl.multiple_of(g * 16, 16)
                iv = idx_v[pl.ds(s, 16)]
                vals = plsc.load_gather(seg_v, [iv])
                out_v[pl.ds(s, 16)] = vals + bvec
                return carry

            lax.fori_loop(0, _GRP, grp, 0, unroll=8)
            pltpu.sync_copy(out_v, out_hbm.at[f, c, pl.ds(h * _HB, _HB)])


def kernel(x, table, bias):
    xT = x.T
    tT = table.T
    biasx = jnp.broadcast_to(bias.T[:, :, None], (_EMB, _NF, 16)).reshape(_EMB, _NF * 16)
    mesh = plsc.VectorSubcoreMesh(core_axis_name="c", subcore_axis_name="s")
    f = pl.kernel(
        _sc_body,
        out_type=jax.ShapeDtypeStruct((_NF, _EMB, _BATCH), jnp.float32),
        mesh=mesh,
        compiler_params=pltpu.CompilerParams(
            use_tc_tiling_on_sc=False,
            needs_layout_passes=False,
        ),
        scratch_types=[
            pltpu.VMEM((_CARD,), jnp.float32),
            pltpu.VMEM((_HB,), jnp.int32),
            pltpu.VMEM((_HB,), jnp.float32),
            pltpu.VMEM((_NF * 16,), jnp.float32),
        ],
    )
    out_p = f(xT, tT, biasx)
    return out_p.transpose(2, 0, 1)
